# baseline (speedup 1.0000x reference)
import functools

import jax
import jax.numpy as jnp
from jax import lax
from jax.experimental import pallas as pl
from jax.experimental.pallas import tpu as pltpu
from jax.experimental.pallas import tpu_sc as plsc

NC = 2
NS = 16
NW = NC * NS
L = 16


@functools.lru_cache(maxsize=None)
def _build_transpose(V: int, D: int):
    assert D == 32
    n_full = V // 128
    tail = V - n_full * 128
    base_t, extra = divmod(n_full, NW)
    mesh = plsc.VectorSubcoreMesh(core_axis_name="c", subcore_axis_name="s")

    NBUF_IN = 5
    NBUF_OUT = 2

    @functools.partial(
        pl.kernel,
        out_type=jax.ShapeDtypeStruct((V * D // 128, 128), jnp.float32),
        mesh=mesh,
        scratch_types=[
            pltpu.VMEM((NBUF_IN * D, 128), jnp.float32),
            pltpu.VMEM((NBUF_OUT * D, 128), jnp.float32),
            pltpu.SemaphoreType.DMA,
            pltpu.SemaphoreType.DMA,
        ],
        compiler_params=pltpu.CompilerParams(use_tc_tiling_on_sc=True,
                                             needs_layout_passes=False,
                                             disable_bounds_checks=True),
    )
    def transpose(wt_hbm, dense_hbm, in_v, out_v, sem_in, sem_out):
        wid = lax.axis_index("s") * NC + lax.axis_index("c")
        n_t = base_t + jnp.where(wid < extra, 1, 0)

        iota = lax.iota(jnp.int32, L)
        rowadd = lax.shift_right_logical(iota, 2)
        coladd = lax.shift_left(jnp.bitwise_and(iota, 3), 5)

        def start_in(t):
            blk = t * NW + wid
            pltpu.async_copy(
                wt_hbm.at[:, pl.ds(blk * 128, 128)],
                in_v.at[pl.ds(lax.rem(t, NBUF_IN) * D, D), :], sem_in)

        def wait_in():
            pltpu.make_async_copy(
                wt_hbm.at[:, pl.ds(0, 128)], in_v.at[pl.ds(0, D), :],
                sem_in).wait()

        def start_out(t):
            blk = t * NW + wid
            pltpu.async_copy(
                out_v.at[pl.ds(lax.rem(t, NBUF_OUT) * D, D), :],
                dense_hbm.at[pl.ds(blk * D, D), :], sem_out)

        def wait_out():
            pltpu.make_async_copy(
                out_v.at[pl.ds(0, D), :], dense_hbm.at[pl.ds(0, D), :],
                sem_out).wait()

        def do_transpose(slot_in, slot_out, ncols):
            for r0 in range(0, ncols, L):
                rows = slot_out * D + (r0 // 4) + rowadd
                for k in range(D):
                    v = in_v[slot_in * D + k, pl.ds(r0, L)]
                    plsc.store_scatter(out_v, [rows, coladd + k], v)

        for tt in range(NBUF_IN):
            start_in(tt)

        @pl.loop(0, base_t)
        def _(t):
            @pl.when(t >= NBUF_OUT)
            def _():
                wait_out()
            wait_in()
            do_transpose(lax.rem(t, NBUF_IN), lax.rem(t, NBUF_OUT), 128)
            start_out(t)

            @pl.when(t + NBUF_IN < n_t)
            def _():
                start_in(t + NBUF_IN)

        @pl.when(wid < extra)
        def _():
            t = base_t
            wait_out()
            wait_in()
            do_transpose(lax.rem(t, NBUF_IN), lax.rem(t, NBUF_OUT), 128)
            start_out(t)

        wait_out()
        wait_out()

        if tail:
            @pl.when(wid == extra)
            def _():
                start = jnp.int32(n_full * 128)
                pltpu.sync_copy(wt_hbm.at[:, pl.ds(start, 128)],
                                in_v.at[pl.ds(0, D), :])
                do_transpose(0, 0, tail)
                pltpu.sync_copy(
                    out_v.at[pl.ds(0, tail * D // 128), :],
                    dense_hbm.at[pl.ds(n_full * D, tail * D // 128), :])

    return transpose


@functools.lru_cache(maxsize=None)
def _build_gather(B: int, V: int, D: int):
    assert B % NW == 0
    b_per_w = B // NW
    chunk = 1600
    while b_per_w % chunk:
        chunk //= 2
    n_chunks = b_per_w // chunk

    mesh = plsc.VectorSubcoreMesh(core_axis_name="c", subcore_axis_name="s")

    @functools.partial(
        pl.kernel,
        out_type=jax.ShapeDtypeStruct((B, 128), jnp.float32),
        mesh=mesh,
        scratch_types=[
            pltpu.VMEM((b_per_w,), jnp.int32),
            pltpu.VMEM((chunk, D), jnp.float32),
            pltpu.VMEM((chunk, D), jnp.float32),
            pltpu.SemaphoreType.DMA,
            pltpu.SemaphoreType.DMA,
            pltpu.SemaphoreType.DMA,
            pltpu.SemaphoreType.DMA,
        ],
        compiler_params=pltpu.CompilerParams(use_tc_tiling_on_sc=False),
    )
    def gather(idx_hbm, table_hbm, out_hbm, idx_all, rows0, rows1,
               sg0, sg1, ss0, ss1):
        wid = lax.axis_index("s") * NC + lax.axis_index("c")
        base = wid * b_per_w
        rows = (rows0, rows1)
        sg = (sg0, sg1)
        ss = (ss0, ss1)

        pltpu.sync_copy(idx_hbm.at[pl.ds(base, b_per_w)], idx_all)

        def idx_slice(j):
            return idx_all.at[pl.ds(j * chunk, chunk)]

        def start_gather(j, b):
            pltpu.async_copy(table_hbm.at[idx_slice(j)], rows[b], sg[b])

        def wait_gather(b):
            pltpu.make_async_copy(table_hbm.at[idx_slice(0)], rows[b],
                                  sg[b]).wait()

        def start_store(j, b):
            pltpu.async_copy(
                rows[b],
                out_hbm.at[pl.ds(base + j * chunk, chunk), pl.ds(0, D)],
                ss[b])

        def wait_store(b):
            pltpu.make_async_copy(
                rows[b], out_hbm.at[pl.ds(base, chunk), pl.ds(0, D)],
                ss[b]).wait()

        start_gather(0, 0)
        start_gather(1, 1)

        @pl.loop(0, n_chunks - 2, step=2)
        def _(i):
            for b in range(2):
                j = i + b
                wait_gather(b)
                start_store(j, b)
                wait_store(b)
                start_gather(j + 2, b)

        for b in range(2):
            j = n_chunks - 2 + b
            wait_gather(b)
            start_store(j, b)
        for b in range(2):
            wait_store(b)

    return gather


def kernel(x, w):
    V, D = w.shape
    x_flat = x.reshape(-1)
    dense = _build_transpose(V, D)(w.T)
    table = dense.reshape(V, D)
    out = _build_gather(x_flat.shape[0], V, D)(x_flat, table)
    return out[:, :D].reshape(x.shape + (D,))


# speedup vs baseline: 1.0305x; 1.0305x over previous
import functools

import jax
import jax.numpy as jnp
from jax import lax
from jax.experimental import pallas as pl
from jax.experimental.pallas import tpu as pltpu
from jax.experimental.pallas import tpu_sc as plsc

NC = 2
NS = 16
NW = NC * NS
L = 16


@functools.lru_cache(maxsize=None)
def _build_transpose(V: int, D: int):
    assert D == 32
    n_full = V // 128
    tail = V - n_full * 128
    base_t, extra = divmod(n_full, NW)
    mesh = plsc.VectorSubcoreMesh(core_axis_name="c", subcore_axis_name="s")

    NBUF_IN = 5
    NBUF_OUT = 2

    @functools.partial(
        pl.kernel,
        out_type=jax.ShapeDtypeStruct((V * D // 128, 128), jnp.float32),
        mesh=mesh,
        scratch_types=[
            pltpu.VMEM((NBUF_IN * D, 128), jnp.float32),
            pltpu.VMEM((NBUF_OUT * D, 128), jnp.float32),
            pltpu.SemaphoreType.DMA,
            pltpu.SemaphoreType.DMA,
        ],
        compiler_params=pltpu.CompilerParams(use_tc_tiling_on_sc=True,
                                             needs_layout_passes=False,
                                             disable_bounds_checks=True),
    )
    def transpose(wt_hbm, dense_hbm, in_v, out_v, sem_in, sem_out):
        wid = lax.axis_index("s") * NC + lax.axis_index("c")
        n_t = base_t + jnp.where(wid < extra, 1, 0)

        iota = lax.iota(jnp.int32, L)
        rowadd = lax.shift_right_logical(iota, 2)
        coladd = lax.shift_left(jnp.bitwise_and(iota, 3), 5)

        def start_in(t):
            blk = t * NW + wid
            pltpu.async_copy(
                wt_hbm.at[:, pl.ds(blk * 128, 128)],
                in_v.at[pl.ds(lax.rem(t, NBUF_IN) * D, D), :], sem_in)

        def wait_in():
            pltpu.make_async_copy(
                wt_hbm.at[:, pl.ds(0, 128)], in_v.at[pl.ds(0, D), :],
                sem_in).wait()

        def start_out(t):
            blk = t * NW + wid
            pltpu.async_copy(
                out_v.at[pl.ds(lax.rem(t, NBUF_OUT) * D, D), :],
                dense_hbm.at[pl.ds(blk * D, D), :], sem_out)

        def wait_out():
            pltpu.make_async_copy(
                out_v.at[pl.ds(0, D), :], dense_hbm.at[pl.ds(0, D), :],
                sem_out).wait()

        def do_transpose(slot_in, slot_out, ncols):
            ngroups = ncols // L
            gbits = ngroups.bit_length() - 1

            @plsc.parallel_loop(0, D * ngroups, unroll=4)
            def _(p):
                k = lax.shift_right_logical(p, gbits)
                r0 = lax.shift_left(jnp.bitwise_and(p, ngroups - 1), 4)
                v = in_v[slot_in * D + k, pl.ds(r0, L)]
                rows = (slot_out * D + lax.shift_right_logical(r0, 2)
                        + rowadd)
                plsc.store_scatter(out_v, [rows, coladd + k], v)

        for tt in range(NBUF_IN):
            start_in(tt)

        @pl.loop(0, base_t)
        def _(t):
            @pl.when(t >= NBUF_OUT)
            def _():
                wait_out()
            wait_in()
            do_transpose(lax.rem(t, NBUF_IN), lax.rem(t, NBUF_OUT), 128)
            start_out(t)

            @pl.when(t + NBUF_IN < n_t)
            def _():
                start_in(t + NBUF_IN)

        @pl.when(wid < extra)
        def _():
            t = base_t
            wait_out()
            wait_in()
            do_transpose(lax.rem(t, NBUF_IN), lax.rem(t, NBUF_OUT), 128)
            start_out(t)

        wait_out()
        wait_out()

        if tail:
            @pl.when(wid == extra)
            def _():
                start = jnp.int32(n_full * 128)
                pltpu.sync_copy(wt_hbm.at[:, pl.ds(start, 128)],
                                in_v.at[pl.ds(0, D), :])
                do_transpose(0, 0, tail)
                pltpu.sync_copy(
                    out_v.at[pl.ds(0, tail * D // 128), :],
                    dense_hbm.at[pl.ds(n_full * D, tail * D // 128), :])

    return transpose


@functools.lru_cache(maxsize=None)
def _build_gather(B: int, V: int, D: int):
    assert B % NW == 0
    b_per_w = B // NW
    chunk = 1600
    while b_per_w % chunk:
        chunk //= 2
    n_chunks = b_per_w // chunk

    mesh = plsc.VectorSubcoreMesh(core_axis_name="c", subcore_axis_name="s")

    @functools.partial(
        pl.kernel,
        out_type=jax.ShapeDtypeStruct((B, 128), jnp.float32),
        mesh=mesh,
        scratch_types=[
            pltpu.VMEM((b_per_w,), jnp.int32),
            pltpu.VMEM((chunk, D), jnp.float32),
            pltpu.VMEM((chunk, D), jnp.float32),
            pltpu.SemaphoreType.DMA,
            pltpu.SemaphoreType.DMA,
            pltpu.SemaphoreType.DMA,
            pltpu.SemaphoreType.DMA,
        ],
        compiler_params=pltpu.CompilerParams(use_tc_tiling_on_sc=False),
    )
    def gather(idx_hbm, table_hbm, out_hbm, idx_all, rows0, rows1,
               sg0, sg1, ss0, ss1):
        wid = lax.axis_index("s") * NC + lax.axis_index("c")
        base = wid * b_per_w
        rows = (rows0, rows1)
        sg = (sg0, sg1)
        ss = (ss0, ss1)

        pltpu.sync_copy(idx_hbm.at[pl.ds(base, b_per_w)], idx_all)

        def idx_slice(j):
            return idx_all.at[pl.ds(j * chunk, chunk)]

        def start_gather(j, b):
            pltpu.async_copy(table_hbm.at[idx_slice(j)], rows[b], sg[b])

        def wait_gather(b):
            pltpu.make_async_copy(table_hbm.at[idx_slice(0)], rows[b],
                                  sg[b]).wait()

        def start_store(j, b):
            pltpu.async_copy(
                rows[b],
                out_hbm.at[pl.ds(base + j * chunk, chunk), pl.ds(0, D)],
                ss[b])

        def wait_store(b):
            pltpu.make_async_copy(
                rows[b], out_hbm.at[pl.ds(base, chunk), pl.ds(0, D)],
                ss[b]).wait()

        start_gather(0, 0)
        start_gather(1, 1)

        @pl.loop(0, n_chunks - 2, step=2)
        def _(i):
            for b in range(2):
                j = i + b
                wait_gather(b)
                start_store(j, b)
                wait_store(b)
                start_gather(j + 2, b)

        for b in range(2):
            j = n_chunks - 2 + b
            wait_gather(b)
            start_store(j, b)
        for b in range(2):
            wait_store(b)

    return gather


def kernel(x, w):
    V, D = w.shape
    x_flat = x.reshape(-1)
    dense = _build_transpose(V, D)(w.T)
    table = dense.reshape(V, D)
    out = _build_gather(x_flat.shape[0], V, D)(x_flat, table)
    return out[:, :D].reshape(x.shape + (D,))


# speedup vs baseline: 1.3342x; 1.2946x over previous
import functools

import jax
import jax.numpy as jnp
from jax import lax
from jax.experimental import pallas as pl
from jax.experimental.pallas import tpu as pltpu
from jax.experimental.pallas import tpu_sc as plsc

NC = 2
NS = 16
NW = NC * NS
L = 16


@functools.lru_cache(maxsize=None)
def _build_tc_transpose(V: int, D: int):
    assert D == 32
    VB = 4096
    steps = -(-V // VB)
    rows_out = V * D // 128

    def body(in_ref, out_ref):
        t = jnp.transpose(in_ref[...], (1, 0))
        t3 = t.reshape(VB // 4, 4, D)
        parts = [t3[:, a, :] for a in range(4)]
        out_ref[...] = jnp.concatenate(parts, axis=1)

    return pl.pallas_call(
        body,
        grid=(steps,),
        in_specs=[pl.BlockSpec((D, VB), lambda j: (0, j))],
        out_specs=pl.BlockSpec((VB * D // 128, 128), lambda j: (j, 0)),
        out_shape=jax.ShapeDtypeStruct((rows_out, 128), jnp.float32),
    )


@functools.lru_cache(maxsize=None)
def _build_transpose(V: int, D: int):
    assert D == 32
    n_full = V // 128
    tail = V - n_full * 128
    base_t, extra = divmod(n_full, NW)
    mesh = plsc.VectorSubcoreMesh(core_axis_name="c", subcore_axis_name="s")

    NBUF_IN = 5
    NBUF_OUT = 2

    @functools.partial(
        pl.kernel,
        out_type=jax.ShapeDtypeStruct((V * D // 128, 128), jnp.float32),
        mesh=mesh,
        scratch_types=[
            pltpu.VMEM((NBUF_IN * D, 128), jnp.float32),
            pltpu.VMEM((NBUF_OUT * D, 128), jnp.float32),
            pltpu.SemaphoreType.DMA,
            pltpu.SemaphoreType.DMA,
        ],
        compiler_params=pltpu.CompilerParams(use_tc_tiling_on_sc=True,
                                             needs_layout_passes=False,
                                             disable_bounds_checks=True),
    )
    def transpose(wt_hbm, dense_hbm, in_v, out_v, sem_in, sem_out):
        wid = lax.axis_index("s") * NC + lax.axis_index("c")
        n_t = base_t + jnp.where(wid < extra, 1, 0)

        iota = lax.iota(jnp.int32, L)
        rowadd = lax.shift_right_logical(iota, 2)
        coladd = lax.shift_left(jnp.bitwise_and(iota, 3), 5)

        def start_in(t):
            blk = t * NW + wid
            pltpu.async_copy(
                wt_hbm.at[:, pl.ds(blk * 128, 128)],
                in_v.at[pl.ds(lax.rem(t, NBUF_IN) * D, D), :], sem_in)

        def wait_in():
            pltpu.make_async_copy(
                wt_hbm.at[:, pl.ds(0, 128)], in_v.at[pl.ds(0, D), :],
                sem_in).wait()

        def start_out(t):
            blk = t * NW + wid
            pltpu.async_copy(
                out_v.at[pl.ds(lax.rem(t, NBUF_OUT) * D, D), :],
                dense_hbm.at[pl.ds(blk * D, D), :], sem_out)

        def wait_out():
            pltpu.make_async_copy(
                out_v.at[pl.ds(0, D), :], dense_hbm.at[pl.ds(0, D), :],
                sem_out).wait()

        def do_transpose(slot_in, slot_out, ncols):
            ngroups = ncols // L
            gbits = ngroups.bit_length() - 1

            @plsc.parallel_loop(0, D * ngroups, unroll=4)
            def _(p):
                k = lax.shift_right_logical(p, gbits)
                r0 = lax.shift_left(jnp.bitwise_and(p, ngroups - 1), 4)
                v = in_v[slot_in * D + k, pl.ds(r0, L)]
                rows = (slot_out * D + lax.shift_right_logical(r0, 2)
                        + rowadd)
                plsc.store_scatter(out_v, [rows, coladd + k], v)

        for tt in range(NBUF_IN):
            start_in(tt)

        @pl.loop(0, base_t)
        def _(t):
            @pl.when(t >= NBUF_OUT)
            def _():
                wait_out()
            wait_in()
            do_transpose(lax.rem(t, NBUF_IN), lax.rem(t, NBUF_OUT), 128)
            start_out(t)

            @pl.when(t + NBUF_IN < n_t)
            def _():
                start_in(t + NBUF_IN)

        @pl.when(wid < extra)
        def _():
            t = base_t
            wait_out()
            wait_in()
            do_transpose(lax.rem(t, NBUF_IN), lax.rem(t, NBUF_OUT), 128)
            start_out(t)

        wait_out()
        wait_out()

        if tail:
            @pl.when(wid == extra)
            def _():
                start = jnp.int32(n_full * 128)
                pltpu.sync_copy(wt_hbm.at[:, pl.ds(start, 128)],
                                in_v.at[pl.ds(0, D), :])
                do_transpose(0, 0, tail)
                pltpu.sync_copy(
                    out_v.at[pl.ds(0, tail * D // 128), :],
                    dense_hbm.at[pl.ds(n_full * D, tail * D // 128), :])

    return transpose


@functools.lru_cache(maxsize=None)
def _build_gather(B: int, V: int, D: int):
    assert B % NW == 0
    b_per_w = B // NW
    chunk = 1600
    while b_per_w % chunk:
        chunk //= 2
    n_chunks = b_per_w // chunk

    mesh = plsc.VectorSubcoreMesh(core_axis_name="c", subcore_axis_name="s")

    @functools.partial(
        pl.kernel,
        out_type=jax.ShapeDtypeStruct((B, 128), jnp.float32),
        mesh=mesh,
        scratch_types=[
            pltpu.VMEM((b_per_w,), jnp.int32),
            pltpu.VMEM((chunk, D), jnp.float32),
            pltpu.VMEM((chunk, D), jnp.float32),
            pltpu.SemaphoreType.DMA,
            pltpu.SemaphoreType.DMA,
            pltpu.SemaphoreType.DMA,
            pltpu.SemaphoreType.DMA,
        ],
        compiler_params=pltpu.CompilerParams(use_tc_tiling_on_sc=False),
    )
    def gather(idx_hbm, table_hbm, out_hbm, idx_all, rows0, rows1,
               sg0, sg1, ss0, ss1):
        wid = lax.axis_index("s") * NC + lax.axis_index("c")
        base = wid * b_per_w
        rows = (rows0, rows1)
        sg = (sg0, sg1)
        ss = (ss0, ss1)

        pltpu.sync_copy(idx_hbm.at[pl.ds(base, b_per_w)], idx_all)

        def idx_slice(j):
            return idx_all.at[pl.ds(j * chunk, chunk)]

        def start_gather(j, b):
            pltpu.async_copy(table_hbm.at[idx_slice(j)], rows[b], sg[b])

        def wait_gather(b):
            pltpu.make_async_copy(table_hbm.at[idx_slice(0)], rows[b],
                                  sg[b]).wait()

        def start_store(j, b):
            pltpu.async_copy(
                rows[b],
                out_hbm.at[pl.ds(base + j * chunk, chunk), pl.ds(0, D)],
                ss[b])

        def wait_store(b):
            pltpu.make_async_copy(
                rows[b], out_hbm.at[pl.ds(base, chunk), pl.ds(0, D)],
                ss[b]).wait()

        start_gather(0, 0)
        start_gather(1, 1)

        @pl.loop(0, n_chunks - 2, step=2)
        def _(i):
            for b in range(2):
                j = i + b
                wait_gather(b)
                start_store(j, b)
                wait_store(b)
                start_gather(j + 2, b)

        for b in range(2):
            j = n_chunks - 2 + b
            wait_gather(b)
            start_store(j, b)
        for b in range(2):
            wait_store(b)

    return gather


def kernel(x, w):
    V, D = w.shape
    x_flat = x.reshape(-1)
    dense = _build_tc_transpose(V, D)(w.T)
    table = dense.reshape(V, D)
    out = _build_gather(x_flat.shape[0], V, D)(x_flat, table)
    return out[:, :D].reshape(x.shape + (D,))
